# baseline (speedup 1.0000x reference)
import functools

import jax
import jax.numpy as jnp
from jax import lax
from jax.experimental import pallas as pl
from jax.experimental.pallas import tpu as pltpu
from jax.experimental.pallas import tpu_sc as plsc

BATCH = 4096
NUM_CLASSES = 1000
PAD_CLASSES = 1024
EMBED_DIM = 16

_SC_INFO = plsc.get_sparse_core_info()
_NC = _SC_INFO.num_cores
_NS = _SC_INFO.num_subcores
_NW = _NC * _NS
_BPW = BATCH // _NW


def _table_body(ft_ref, prm_ref, out_ref):
    acc = jnp.zeros((1, PAD_CLASSES), jnp.float32)
    base = prm_ref[2, 0]
    for d in range(EMBED_DIM):
        e0d = prm_ref[0, d]
        e1d = prm_ref[1, d]
        fd = ft_ref[d]
        acc = acc + fd * (e1d - e0d)
        base = base + e0d * jnp.sum(fd)
    out_ref[...] = (acc + base).reshape(PAD_CLASSES)


def _lookup_kernel(table_hbm, idx_hbm, out_hbm, idx_v, out_v, sem):
    wid = lax.axis_index("s") * _NC + lax.axis_index("c")
    start = wid * _BPW
    pltpu.sync_copy(idx_hbm.at[pl.ds(start, _BPW)], idx_v)
    pltpu.async_copy(table_hbm.at[idx_v], out_v, sem).wait()
    pltpu.sync_copy(out_v, out_hbm.at[pl.ds(start, _BPW)])


@jax.jit
def kernel(sample_loc, embed_weight, fc_w, fc_b):
    fc_flat = fc_w.reshape(-1).astype(jnp.float32)
    fc_pad = jnp.concatenate(
        [fc_flat, jnp.zeros((PAD_CLASSES * EMBED_DIM - fc_flat.shape[0],), jnp.float32)]
    )
    fcb = fc_pad.astype(jnp.bfloat16).astype(jnp.float32)
    ft = fcb.reshape(PAD_CLASSES, EMBED_DIM).T.reshape(EMBED_DIM, 1, PAD_CLASSES)

    e01b = embed_weight[0:2].astype(jnp.bfloat16).astype(jnp.float32)
    params = jnp.stack(
        [
            e01b[0],
            e01b[1],
            jnp.full((EMBED_DIM,), fc_b[0], jnp.float32),
            jnp.zeros((EMBED_DIM,), jnp.float32),
        ]
    )

    table = pl.pallas_call(
        _table_body,
        out_shape=jax.ShapeDtypeStruct((PAD_CLASSES,), jnp.float32),
        in_specs=[
            pl.BlockSpec(memory_space=pltpu.VMEM),
            pl.BlockSpec(memory_space=pltpu.VMEM),
        ],
        out_specs=pl.BlockSpec(memory_space=pltpu.VMEM),
    )(ft, params)

    idx = sample_loc.astype(jnp.int32)

    mesh = plsc.VectorSubcoreMesh(core_axis_name="c", subcore_axis_name="s")
    lookup = functools.partial(
        pl.kernel,
        mesh=mesh,
        out_type=jax.ShapeDtypeStruct((BATCH,), jnp.float32),
        scratch_types=[
            pltpu.VMEM((_BPW,), jnp.int32),
            pltpu.VMEM((_BPW,), jnp.float32),
            pltpu.SemaphoreType.DMA,
        ],
    )(_lookup_kernel)
    out = lookup(table, idx)
    return out.reshape(BATCH, 1)


# speedup vs baseline: 1.0020x; 1.0020x over previous
import functools

import jax
import jax.numpy as jnp
from jax import lax
from jax.experimental import pallas as pl
from jax.experimental.pallas import tpu as pltpu
from jax.experimental.pallas import tpu_sc as plsc

BATCH = 4096
NUM_CLASSES = 1000
PAD_CLASSES = 1024
EMBED_DIM = 16

_SC_INFO = plsc.get_sparse_core_info()
_NC = _SC_INFO.num_cores
_NS = _SC_INFO.num_subcores
_NW = _NC * _NS
_BPW = BATCH // _NW


def _table_body(fc2_ref, e01_ref, b_ref, out_ref):
    fcb = fc2_ref[...].astype(jnp.bfloat16)
    e0b = e01_ref[0:1, :].astype(jnp.bfloat16)
    e1b = e01_ref[1:2, :].astype(jnp.bfloat16)
    dn = (((1,), (1,)), ((), ()))
    t0 = lax.dot_general(e0b, fcb, dn, preferred_element_type=jnp.float32)
    t1 = lax.dot_general(e1b, fcb, dn, preferred_element_type=jnp.float32)
    base = jnp.sum(t0) + b_ref[0, 0]
    out_ref[...] = (t1 - t0 + base).reshape(NUM_CLASSES)


def _lookup_kernel(table_hbm, idx_hbm, out_hbm, idx_v, out_v, sem):
    wid = lax.axis_index("s") * _NC + lax.axis_index("c")
    start = wid * _BPW
    pltpu.sync_copy(idx_hbm.at[pl.ds(start, _BPW)], idx_v)
    pltpu.async_copy(table_hbm.at[idx_v], out_v, sem).wait()
    pltpu.sync_copy(out_v, out_hbm.at[pl.ds(start, _BPW)])


@jax.jit
def kernel(sample_loc, embed_weight, fc_w, fc_b):
    fc2 = fc_w.reshape(NUM_CLASSES, EMBED_DIM).astype(jnp.float32)
    e01 = embed_weight[0:2].astype(jnp.float32)
    b2d = fc_b.reshape(1, 1).astype(jnp.float32)

    table = pl.pallas_call(
        _table_body,
        out_shape=jax.ShapeDtypeStruct((NUM_CLASSES,), jnp.float32),
        in_specs=[
            pl.BlockSpec(memory_space=pltpu.VMEM),
            pl.BlockSpec(memory_space=pltpu.VMEM),
            pl.BlockSpec(memory_space=pltpu.VMEM),
        ],
        out_specs=pl.BlockSpec(memory_space=pltpu.VMEM),
    )(fc2, e01, b2d)

    idx = sample_loc.astype(jnp.int32)

    mesh = plsc.VectorSubcoreMesh(core_axis_name="c", subcore_axis_name="s")
    lookup = functools.partial(
        pl.kernel,
        mesh=mesh,
        out_type=jax.ShapeDtypeStruct((BATCH,), jnp.float32),
        scratch_types=[
            pltpu.VMEM((_BPW,), jnp.int32),
            pltpu.VMEM((_BPW,), jnp.float32),
            pltpu.SemaphoreType.DMA,
        ],
    )(_lookup_kernel)
    out = lookup(table, idx)
    return out.reshape(BATCH, 1)


# speedup vs baseline: 1.0118x; 1.0098x over previous
import functools

import jax
import jax.numpy as jnp
from jax import lax
from jax.experimental import pallas as pl
from jax.experimental.pallas import tpu as pltpu
from jax.experimental.pallas import tpu_sc as plsc

BATCH = 4096
NUM_CLASSES = 1000
PAD_CLASSES = 1024
EMBED_DIM = 16

_SC_INFO = plsc.get_sparse_core_info()
_NC = _SC_INFO.num_cores
_NS = _SC_INFO.num_subcores
_NW = _NC * _NS
_BPW = BATCH // _NW


def _table_body(fc2_ref, e01_ref, b_ref, out_ref):
    fcb = fc2_ref[...].astype(jnp.bfloat16)
    e0b = e01_ref[0:1, :].astype(jnp.bfloat16)
    e1b = e01_ref[1:2, :].astype(jnp.bfloat16)
    dn = (((1,), (1,)), ((), ()))
    t0 = lax.dot_general(e0b, fcb, dn, preferred_element_type=jnp.float32)
    t1 = lax.dot_general(e1b, fcb, dn, preferred_element_type=jnp.float32)
    base = jnp.sum(t0) + b_ref[0]
    out_ref[...] = (t1 - t0 + base).reshape(NUM_CLASSES)


def _lookup_kernel(table_hbm, idx_hbm, out_hbm, idx_v, out_v, sem):
    wid = lax.axis_index("s") * _NC + lax.axis_index("c")
    start = wid * _BPW
    pltpu.sync_copy(idx_hbm.at[pl.ds(start, _BPW)], idx_v)
    pltpu.async_copy(table_hbm.at[idx_v], out_v, sem).wait()
    pltpu.sync_copy(out_v, out_hbm.at[pl.ds(start, _BPW)])


@jax.jit
def kernel(sample_loc, embed_weight, fc_w, fc_b):
    table = pl.pallas_call(
        _table_body,
        out_shape=jax.ShapeDtypeStruct((NUM_CLASSES,), jnp.float32),
        in_specs=[
            pl.BlockSpec(memory_space=pltpu.VMEM),
            pl.BlockSpec(memory_space=pltpu.VMEM),
            pl.BlockSpec(memory_space=pltpu.VMEM),
        ],
        out_specs=pl.BlockSpec(memory_space=pltpu.VMEM),
    )(fc_w.astype(jnp.float32).reshape(NUM_CLASSES, EMBED_DIM),
      embed_weight.astype(jnp.float32), fc_b.astype(jnp.float32))

    idx = sample_loc.astype(jnp.int32)

    mesh = plsc.VectorSubcoreMesh(core_axis_name="c", subcore_axis_name="s")
    lookup = functools.partial(
        pl.kernel,
        mesh=mesh,
        out_type=jax.ShapeDtypeStruct((BATCH,), jnp.float32),
        scratch_types=[
            pltpu.VMEM((_BPW,), jnp.int32),
            pltpu.VMEM((_BPW,), jnp.float32),
            pltpu.SemaphoreType.DMA,
        ],
    )(_lookup_kernel)
    out = lookup(table, idx)
    return out.reshape(BATCH, 1)
